# baseline (speedup 1.0000x reference)
import functools

import jax
import jax.numpy as jnp
import numpy as np
from jax import lax
from jax.experimental import pallas as pl
from jax.experimental.pallas import tpu as pltpu
from jax.experimental.pallas import tpu_sc as plsc

_FEATURE_DIMS = [1000] * 26
_F = 26
_D = 16
_V = 26000
_B = 4096
_P = _F * (_F - 1) // 2
_IXW = _P * _D
_LINC = _IXW
_AUGW = 5376
_TABW = 512

_NC, _NS = 2, 16
_NW = _NC * _NS
_NB = _B // _NW
_CB = 4
_NCHUNK = _NB // _CB
_ROWS = _CB * _F

_BT = 256
_H = 400


def _sc_gather_interact(xo_hbm, tab_hbm, feat_hbm, idx_v, rows_v, ix_v, sem):
    w = lax.axis_index("s") * _NC + lax.axis_index("c")

    zero = jnp.zeros((_D,), jnp.float32)
    for bb in range(_CB):
        for col in range(_LINC + _D, _AUGW, _D):
            ix_v[bb, pl.ds(col, _D)] = zero

    def chunk_body(c, carry):
        base_b = w * _NB + c * _CB
        pltpu.sync_copy(xo_hbm.at[pl.ds(base_b * _F, _ROWS)], idx_v)
        pltpu.async_copy(tab_hbm.at[idx_v], rows_v, sem).wait()

        for bb in range(_CB):
            rbase = bb * _F
            lv = rows_v[rbase, pl.ds(_F * _D, _D)]
            for i in range(1, _F):
                lv = lv + rows_v[rbase + i, pl.ds(_F * _D, _D)]
            ix_v[bb, pl.ds(_LINC, _D)] = lv
            p = 0
            for i in range(_F):
                for j in range(i + 1, _F):
                    a = rows_v[rbase + i, pl.ds(j * _D, _D)]
                    b = rows_v[rbase + j, pl.ds(i * _D, _D)]
                    ix_v[bb, pl.ds(p * _D, _D)] = a * b
                    p += 1
        pltpu.sync_copy(ix_v, feat_hbm.at[pl.ds(base_b, _CB)])
        return carry

    lax.fori_loop(0, _NCHUNK, chunk_body, 0)


_sc_features = functools.partial(
    pl.kernel,
    out_type=jax.ShapeDtypeStruct((_B, _AUGW), jnp.float32),
    mesh=plsc.VectorSubcoreMesh(core_axis_name="c", subcore_axis_name="s"),
    scratch_types=[
        pltpu.VMEM((_ROWS,), jnp.int32),
        pltpu.VMEM((_ROWS, _TABW), jnp.float32),
        pltpu.VMEM((_CB, _AUGW), jnp.float32),
        pltpu.SemaphoreType.DMA,
    ],
)(_sc_gather_interact)


def _tc_mlp(feat_ref, w1_ref, b1_ref, w2_ref, b2_ref, w3_ref, b3_ref,
            el_ref, out_ref):
    x = feat_ref[...]
    xb = x.astype(jnp.bfloat16)
    h = jnp.dot(xb, w1_ref[...], preferred_element_type=jnp.float32)
    h = jnp.maximum(h + b1_ref[...], 0.0)
    h = jnp.dot(h.astype(jnp.bfloat16), w2_ref[...],
                preferred_element_type=jnp.float32)
    h = jnp.maximum(h + b2_ref[...], 0.0)
    d = jnp.dot(h.astype(jnp.bfloat16), w3_ref[...],
                preferred_element_type=jnp.float32)
    lin = jnp.dot(x, el_ref[...], preferred_element_type=jnp.float32)
    z = d + lin + b3_ref[...]
    out_ref[...] = 1.0 / (1.0 + jnp.exp(-z))


def kernel(x, linear_w, linear_b, ffm_tables, W1, b1, W2, b2, W3, b3):
    offsets = jnp.asarray(
        np.concatenate([[0], np.cumsum(_FEATURE_DIMS)[:-1]]), dtype=x.dtype)
    xo = (x + offsets[None, :]).reshape(-1)

    tab = jnp.transpose(ffm_tables, (1, 0, 2)).reshape(_V, _F * _D)
    tab = jnp.concatenate(
        [tab, linear_w.reshape(_V, 1),
         jnp.zeros((_V, _TABW - _F * _D - 1), jnp.float32)], axis=1)

    feat = _sc_features(xo, tab)

    w1p = jnp.concatenate(
        [W1, jnp.zeros((_AUGW - _IXW, _H), jnp.float32)],
        axis=0).astype(jnp.bfloat16)
    el = jnp.zeros((_AUGW, 1), jnp.float32).at[_LINC:_LINC + _D].set(1.0)
    b3c = (b3 + linear_b).reshape(1, 1)

    out2d = pl.pallas_call(
        _tc_mlp,
        grid=(_B // _BT,),
        in_specs=[
            pl.BlockSpec((_BT, _AUGW), lambda i: (i, 0)),
            pl.BlockSpec((_AUGW, _H), lambda i: (0, 0)),
            pl.BlockSpec((1, _H), lambda i: (0, 0)),
            pl.BlockSpec((_H, _H), lambda i: (0, 0)),
            pl.BlockSpec((1, _H), lambda i: (0, 0)),
            pl.BlockSpec((_H, 1), lambda i: (0, 0)),
            pl.BlockSpec((1, 1), lambda i: (0, 0)),
            pl.BlockSpec((_AUGW, 1), lambda i: (0, 0)),
        ],
        out_specs=pl.BlockSpec((_BT, 1), lambda i: (i, 0)),
        out_shape=jax.ShapeDtypeStruct((_B, 1), jnp.float32),
        compiler_params=pltpu.CompilerParams(
            dimension_semantics=("arbitrary",)),
    )(feat, w1p, b1.reshape(1, _H), W2.astype(jnp.bfloat16),
      b2.reshape(1, _H), W3.astype(jnp.bfloat16), b3c, el)

    return out2d.reshape(_B)


# speedup vs baseline: 1.0398x; 1.0398x over previous
import functools

import jax
import jax.numpy as jnp
import numpy as np
from jax import lax
from jax.experimental import pallas as pl
from jax.experimental.pallas import tpu as pltpu
from jax.experimental.pallas import tpu_sc as plsc

_FEATURE_DIMS = [1000] * 26
_F = 26
_D = 16
_V = 26000
_B = 4096
_P = _F * (_F - 1) // 2
_IXW = _P * _D
_LINC = _IXW
_AUGW = 5376
_TABW = 512

_NC, _NS = 2, 16
_NW = _NC * _NS
_NB = _B // _NW
_CB = 4
_NCHUNK = _NB // _CB
_ROWS = _CB * _F

_BT = 256
_H = 400


def _sc_gather_interact(xo_hbm, tab_hbm, feat_hbm, idx_v, rows_v, ix_v, sem):
    w = lax.axis_index("s") * _NC + lax.axis_index("c")

    pltpu.sync_copy(xo_hbm.at[pl.ds(w * (_NB * _F), _NB * _F)], idx_v)

    zero = jnp.zeros((_D,), jnp.float32)
    for bb in range(_CB):
        for col in range(_LINC + _D, _AUGW, _D):
            ix_v[bb, pl.ds(col, _D)] = zero

    def chunk_body(c, carry):
        base_b = w * _NB + c * _CB
        ib = pl.multiple_of(c * _ROWS, 8)
        pltpu.async_copy(tab_hbm.at[idx_v.at[pl.ds(ib, _ROWS)]],
                         rows_v, sem).wait()

        def batch_body(bb, carry2):
            rbase = bb * _F
            lv = rows_v[rbase, pl.ds(_F * _D, _D)]
            for i in range(1, _F):
                lv = lv + rows_v[rbase + i, pl.ds(_F * _D, _D)]
            ix_v[bb, pl.ds(_LINC, _D)] = lv
            p = 0
            for i in range(_F):
                for j in range(i + 1, _F):
                    a = rows_v[rbase + i, pl.ds(j * _D, _D)]
                    b = rows_v[rbase + j, pl.ds(i * _D, _D)]
                    ix_v[bb, pl.ds(p * _D, _D)] = a * b
                    p += 1
            return carry2

        lax.fori_loop(0, _CB, batch_body, 0)
        pltpu.sync_copy(ix_v, feat_hbm.at[pl.ds(base_b, _CB)])
        return carry

    lax.fori_loop(0, _NCHUNK, chunk_body, 0)


_sc_features = functools.partial(
    pl.kernel,
    out_type=jax.ShapeDtypeStruct((_B, _AUGW), jnp.float32),
    mesh=plsc.VectorSubcoreMesh(core_axis_name="c", subcore_axis_name="s"),
    scratch_types=[
        pltpu.VMEM((_NB * _F,), jnp.int32),
        pltpu.VMEM((_ROWS, _TABW), jnp.float32),
        pltpu.VMEM((_CB, _AUGW), jnp.float32),
        pltpu.SemaphoreType.DMA,
    ],
)(_sc_gather_interact)


def _tc_mlp(feat_ref, w1_ref, b1_ref, w2_ref, b2_ref, w3_ref, b3_ref,
            el_ref, out_ref):
    x = feat_ref[...]
    xb = x.astype(jnp.bfloat16)
    h = jnp.dot(xb, w1_ref[...], preferred_element_type=jnp.float32)
    h = jnp.maximum(h + b1_ref[...], 0.0)
    h = jnp.dot(h.astype(jnp.bfloat16), w2_ref[...],
                preferred_element_type=jnp.float32)
    h = jnp.maximum(h + b2_ref[...], 0.0)
    d = jnp.dot(h.astype(jnp.bfloat16), w3_ref[...],
                preferred_element_type=jnp.float32)
    lin = jnp.dot(x, el_ref[...], preferred_element_type=jnp.float32)
    z = d + lin + b3_ref[...]
    out_ref[...] = 1.0 / (1.0 + jnp.exp(-z))


def kernel(x, linear_w, linear_b, ffm_tables, W1, b1, W2, b2, W3, b3):
    offsets = jnp.asarray(
        np.concatenate([[0], np.cumsum(_FEATURE_DIMS)[:-1]]), dtype=x.dtype)
    xo = (x + offsets[None, :]).reshape(-1)

    tab = jnp.transpose(ffm_tables, (1, 0, 2)).reshape(_V, _F * _D)
    tab = jnp.concatenate(
        [tab, linear_w.reshape(_V, 1),
         jnp.zeros((_V, _TABW - _F * _D - 1), jnp.float32)], axis=1)

    feat = _sc_features(xo, tab)

    w1p = jnp.concatenate(
        [W1, jnp.zeros((_AUGW - _IXW, _H), jnp.float32)],
        axis=0).astype(jnp.bfloat16)
    el = jnp.zeros((_AUGW, 1), jnp.float32).at[_LINC:_LINC + _D].set(1.0)
    b3c = (b3 + linear_b).reshape(1, 1)

    out2d = pl.pallas_call(
        _tc_mlp,
        grid=(_B // _BT,),
        in_specs=[
            pl.BlockSpec((_BT, _AUGW), lambda i: (i, 0)),
            pl.BlockSpec((_AUGW, _H), lambda i: (0, 0)),
            pl.BlockSpec((1, _H), lambda i: (0, 0)),
            pl.BlockSpec((_H, _H), lambda i: (0, 0)),
            pl.BlockSpec((1, _H), lambda i: (0, 0)),
            pl.BlockSpec((_H, 1), lambda i: (0, 0)),
            pl.BlockSpec((1, 1), lambda i: (0, 0)),
            pl.BlockSpec((_AUGW, 1), lambda i: (0, 0)),
        ],
        out_specs=pl.BlockSpec((_BT, 1), lambda i: (i, 0)),
        out_shape=jax.ShapeDtypeStruct((_B, 1), jnp.float32),
        compiler_params=pltpu.CompilerParams(
            dimension_semantics=("arbitrary",)),
    )(feat, w1p, b1.reshape(1, _H), W2.astype(jnp.bfloat16),
      b2.reshape(1, _H), W3.astype(jnp.bfloat16), b3c, el)

    return out2d.reshape(_B)


# speedup vs baseline: 1.0521x; 1.0119x over previous
import functools

import jax
import jax.numpy as jnp
import numpy as np
from jax import lax
from jax.experimental import pallas as pl
from jax.experimental.pallas import tpu as pltpu
from jax.experimental.pallas import tpu_sc as plsc

_FEATURE_DIMS = [1000] * 26
_F = 26
_D = 16
_V = 26000
_B = 4096
_P = _F * (_F - 1) // 2
_IXW = _P * _D
_LINC = _IXW
_AUGW = 5376
_TABW = 512

_NC, _NS = 2, 16
_NW = _NC * _NS
_BH = _B // 2
_NB = _BH // _NW
_CB = 4
_NCHUNK = _NB // _CB
_ROWS = _CB * _F

_BT = 256
_H = 400


def _sc_gather_interact(xo_hbm, tab_hbm, feat_hbm, idx_v, rows_v, ix_v, sem):
    w = lax.axis_index("s") * _NC + lax.axis_index("c")

    pltpu.sync_copy(xo_hbm.at[pl.ds(w * (_NB * _F), _NB * _F)], idx_v)

    zero = jnp.zeros((_D,), jnp.float32)
    for bb in range(_CB):
        for col in range(_LINC + _D, _AUGW, _D):
            ix_v[bb, pl.ds(col, _D)] = zero

    def chunk_body(c, carry):
        base_b = w * _NB + c * _CB
        ib = pl.multiple_of(c * _ROWS, 8)
        pltpu.async_copy(tab_hbm.at[idx_v.at[pl.ds(ib, _ROWS)]],
                         rows_v, sem).wait()

        def batch_body(bb, carry2):
            rbase = bb * _F
            lv = rows_v[rbase, pl.ds(_F * _D, _D)]
            for i in range(1, _F):
                lv = lv + rows_v[rbase + i, pl.ds(_F * _D, _D)]
            ix_v[bb, pl.ds(_LINC, _D)] = lv
            p = 0
            for i in range(_F):
                for j in range(i + 1, _F):
                    a = rows_v[rbase + i, pl.ds(j * _D, _D)]
                    b = rows_v[rbase + j, pl.ds(i * _D, _D)]
                    ix_v[bb, pl.ds(p * _D, _D)] = a * b
                    p += 1
            return carry2

        lax.fori_loop(0, _CB, batch_body, 0)
        pltpu.sync_copy(ix_v, feat_hbm.at[pl.ds(base_b, _CB)])
        return carry

    lax.fori_loop(0, _NCHUNK, chunk_body, 0)


_sc_features = functools.partial(
    pl.kernel,
    out_type=jax.ShapeDtypeStruct((_BH, _AUGW), jnp.float32),
    mesh=plsc.VectorSubcoreMesh(core_axis_name="c", subcore_axis_name="s"),
    scratch_types=[
        pltpu.VMEM((_NB * _F,), jnp.int32),
        pltpu.VMEM((_ROWS, _TABW), jnp.float32),
        pltpu.VMEM((_CB, _AUGW), jnp.float32),
        pltpu.SemaphoreType.DMA,
    ],
)(_sc_gather_interact)


def _tc_mlp(feat_ref, w1_ref, b1_ref, w2_ref, b2_ref, w3_ref, b3_ref,
            el_ref, out_ref):
    x = feat_ref[...]
    xb = x.astype(jnp.bfloat16)
    h = jnp.dot(xb, w1_ref[...], preferred_element_type=jnp.float32)
    h = jnp.maximum(h + b1_ref[...], 0.0)
    h = jnp.dot(h.astype(jnp.bfloat16), w2_ref[...],
                preferred_element_type=jnp.float32)
    h = jnp.maximum(h + b2_ref[...], 0.0)
    d = jnp.dot(h.astype(jnp.bfloat16), w3_ref[...],
                preferred_element_type=jnp.float32)
    lin = jnp.dot(x, el_ref[...], preferred_element_type=jnp.float32)
    z = d + lin + b3_ref[...]
    out_ref[...] = 1.0 / (1.0 + jnp.exp(-z))


def kernel(x, linear_w, linear_b, ffm_tables, W1, b1, W2, b2, W3, b3):
    offsets = jnp.asarray(
        np.concatenate([[0], np.cumsum(_FEATURE_DIMS)[:-1]]), dtype=x.dtype)
    xo = (x + offsets[None, :]).reshape(-1)

    tab = jnp.transpose(ffm_tables, (1, 0, 2)).reshape(_V, _F * _D)
    tab = jnp.concatenate(
        [tab, linear_w.reshape(_V, 1),
         jnp.zeros((_V, _TABW - _F * _D - 1), jnp.float32)], axis=1)

    feat1 = _sc_features(xo[:_BH * _F], tab)
    feat2 = _sc_features(xo[_BH * _F:], tab)

    w1p = jnp.concatenate(
        [W1, jnp.zeros((_AUGW - _IXW, _H), jnp.float32)],
        axis=0).astype(jnp.bfloat16)
    el = jnp.zeros((_AUGW, 1), jnp.float32).at[_LINC:_LINC + _D].set(1.0)
    b3c = (b3 + linear_b).reshape(1, 1)

    mlp = pl.pallas_call(
        _tc_mlp,
        grid=(_BH // _BT,),
        in_specs=[
            pl.BlockSpec((_BT, _AUGW), lambda i: (i, 0)),
            pl.BlockSpec((_AUGW, _H), lambda i: (0, 0)),
            pl.BlockSpec((1, _H), lambda i: (0, 0)),
            pl.BlockSpec((_H, _H), lambda i: (0, 0)),
            pl.BlockSpec((1, _H), lambda i: (0, 0)),
            pl.BlockSpec((_H, 1), lambda i: (0, 0)),
            pl.BlockSpec((1, 1), lambda i: (0, 0)),
            pl.BlockSpec((_AUGW, 1), lambda i: (0, 0)),
        ],
        out_specs=pl.BlockSpec((_BT, 1), lambda i: (i, 0)),
        out_shape=jax.ShapeDtypeStruct((_BH, 1), jnp.float32),
        compiler_params=pltpu.CompilerParams(
            dimension_semantics=("arbitrary",)),
    )
    args = (w1p, b1.reshape(1, _H), W2.astype(jnp.bfloat16),
            b2.reshape(1, _H), W3.astype(jnp.bfloat16), b3c, el)
    out1 = mlp(feat1, *args)
    out2 = mlp(feat2, *args)

    return jnp.concatenate([out1, out2], axis=0).reshape(_B)
